# Initial kernel scaffold; baseline (speedup 1.0000x reference)
import jax
import jax.numpy as jnp
from jax.experimental import pallas as pl


def kernel(inputs, embedding):
    raise NotImplementedError("write your pallas kernel here")


# speedup vs baseline: 1.3077x; 1.3077x over previous
import jax
import jax.numpy as jnp
from jax import lax
from jax.experimental import pallas as pl
from jax.experimental.pallas import tpu as pltpu
from jax.experimental.pallas import tpu_sc as plsc

NUM_CORES = 2
NUM_SUBCORES = 16
NUM_WORKERS = NUM_CORES * NUM_SUBCORES
CHUNK = 128


def _build(n_rows, feat, cpw):
    def body(table_hbm, idx_hbm, out_hbm, idx_v, buf_v, gsem):
        wid = lax.axis_index("s") * NUM_CORES + lax.axis_index("c")
        base = wid * cpw
        pltpu.sync_copy(idx_hbm.at[pl.ds(base, cpw)], idx_v)

        @pl.loop(0, cpw)
        def _(j):
            pltpu.async_copy(table_hbm.at[idx_v.at[j]], buf_v, gsem).wait()
            pltpu.sync_copy(buf_v, out_hbm.at[pl.ds((base + j) * CHUNK, CHUNK)])

    return pl.kernel(
        body,
        out_type=jax.ShapeDtypeStruct((n_rows, feat), jnp.float32),
        mesh=plsc.VectorSubcoreMesh(core_axis_name="c", subcore_axis_name="s"),
        scratch_types=[
            pltpu.VMEM((cpw, CHUNK), jnp.int32),
            pltpu.VMEM((CHUNK, feat), jnp.float32),
            pltpu.SemaphoreType.DMA,
        ],
        compiler_params=pltpu.CompilerParams(use_tc_tiling_on_sc=False),
    )


def kernel(inputs, embedding):
    batch, seq = inputs.shape
    _, feat = embedding.shape
    n_rows = batch * seq
    n_chunks = n_rows // CHUNK
    cpw = n_chunks // NUM_WORKERS
    idx2 = inputs.reshape(n_chunks, CHUNK).astype(jnp.int32)
    out = _build(n_rows, feat, cpw)(embedding, idx2)
    return out.reshape(batch, seq, feat)


# speedup vs baseline: 1.5015x; 1.1482x over previous
import jax
import jax.numpy as jnp
from jax import lax
from jax.experimental import pallas as pl
from jax.experimental.pallas import tpu as pltpu
from jax.experimental.pallas import tpu_sc as plsc

NUM_CORES = 2
NUM_SUBCORES = 16
NUM_WORKERS = NUM_CORES * NUM_SUBCORES
CHUNK = 128
K = 4
NBUF = 5


def _build(n_rows, feat, cpw):
    ns = cpw // K

    def body(table_hbm, idx_hbm, out_hbm, idx_v, bufs, gsems, ssems):
        wid = lax.axis_index("s") * NUM_CORES + lax.axis_index("c")
        base = wid * cpw
        pltpu.sync_copy(idx_hbm.at[pl.ds(base, cpw)], idx_v)

        def fire_gathers(s, b):
            for g in range(K):
                pltpu.async_copy(
                    table_hbm.at[idx_v.at[s * K + g]],
                    bufs.at[b, pl.ds(g * CHUNK, CHUNK)],
                    gsems.at[b],
                )

        def drain_gathers(b):
            pltpu.make_async_copy(
                table_hbm.at[pl.ds(0, K * CHUNK)], bufs.at[b], gsems.at[b]
            ).wait()

        def fire_store(s, b):
            pltpu.async_copy(
                bufs.at[b],
                out_hbm.at[pl.ds((base + s * K) * CHUNK, K * CHUNK)],
                ssems.at[b],
            )

        def wait_store(b):
            pltpu.make_async_copy(
                bufs.at[b], out_hbm.at[pl.ds(0, K * CHUNK)], ssems.at[b]
            ).wait()

        for s in range(3):
            fire_gathers(s, s)
        for s in range(2):
            drain_gathers(s)
            fire_store(s, s)
            fire_gathers(s + 3, s + 3)

        @pl.loop(0, ns - 5, step=NBUF)
        def _(t):
            for b in range(NBUF):
                s = t + 2 + b
                bb = (2 + b) % NBUF
                drain_gathers(bb)
                fire_store(s, bb)
                wait_store(b)
                fire_gathers(s + 3, b)

        for i in range(3):
            s = ns - 3 + i
            drain_gathers(s % NBUF)
            fire_store(s, s % NBUF)
        for i in range(NBUF):
            wait_store((ns - NBUF + i) % NBUF)

    return pl.kernel(
        body,
        out_type=jax.ShapeDtypeStruct((n_rows, feat), jnp.float32),
        mesh=plsc.VectorSubcoreMesh(core_axis_name="c", subcore_axis_name="s"),
        scratch_types=[
            pltpu.VMEM((cpw, CHUNK), jnp.int32),
            pltpu.VMEM((NBUF, K * CHUNK, feat), jnp.float32),
            pltpu.SemaphoreType.DMA((NBUF,)),
            pltpu.SemaphoreType.DMA((NBUF,)),
        ],
        compiler_params=pltpu.CompilerParams(use_tc_tiling_on_sc=False),
    )


def kernel(inputs, embedding):
    batch, seq = inputs.shape
    _, feat = embedding.shape
    n_rows = batch * seq
    n_chunks = n_rows // CHUNK
    cpw = n_chunks // NUM_WORKERS
    idx2 = inputs.reshape(n_chunks, CHUNK).astype(jnp.int32)
    out = _build(n_rows, feat, cpw)(embedding, idx2)
    return out.reshape(batch, seq, feat)


# speedup vs baseline: 1.5021x; 1.0004x over previous
import jax
import jax.numpy as jnp
from jax import lax
from jax.experimental import pallas as pl
from jax.experimental.pallas import tpu as pltpu
from jax.experimental.pallas import tpu_sc as plsc

NUM_CORES = 2
NUM_SUBCORES = 16
NUM_WORKERS = NUM_CORES * NUM_SUBCORES
SPLITS = ((0, 120), (120, 80))
NBS = 2
NBUF = 4


def _build(batch, seq, feat):
    bpw = batch // NUM_WORKERS
    ns = bpw // NBS
    assert (ns - 4) % NBUF == 0 and ns >= 8

    def body(idx_hbm, table_hbm, out_hbm, idx_v, bufs, gsems, ssems):
        wid = lax.axis_index("s") * NUM_CORES + lax.axis_index("c")
        b0 = wid * bpw
        pltpu.sync_copy(idx_hbm.at[pl.ds(b0, bpw)], idx_v)

        def fire_gathers(s, bb):
            for r in range(NBS):
                j = s * NBS + r
                for off, ln in SPLITS:
                    pltpu.async_copy(
                        table_hbm.at[idx_v.at[j, pl.ds(off, ln)]],
                        bufs.at[bb, r, pl.ds(off, ln)],
                        gsems.at[bb],
                    )

        def drain_gathers(bb):
            pltpu.make_async_copy(
                out_hbm.at[pl.ds(0, NBS)], bufs.at[bb], gsems.at[bb]
            ).wait()

        def fire_store(s, bb):
            pltpu.async_copy(
                bufs.at[bb], out_hbm.at[pl.ds(b0 + s * NBS, NBS)], ssems.at[bb]
            )

        def wait_store(bb):
            pltpu.make_async_copy(
                bufs.at[bb], out_hbm.at[pl.ds(0, NBS)], ssems.at[bb]
            ).wait()

        for s in range(3):
            fire_gathers(s, s)
        drain_gathers(0)
        fire_store(0, 0)
        fire_gathers(3, 3)

        @pl.loop(0, ns - 4, step=NBUF)
        def _(t):
            for b in range(NBUF):
                s = t + 1 + b
                bb = (1 + b) % NBUF
                drain_gathers(bb)
                fire_store(s, bb)
                nb = (bb + 3) % NBUF
                wait_store(nb)
                fire_gathers(s + 3, nb)

        for i in range(3):
            s = ns - 3 + i
            drain_gathers(s % NBUF)
            fire_store(s, s % NBUF)
        for i in range(NBUF):
            wait_store((ns - NBUF + i) % NBUF)

    return pl.kernel(
        body,
        out_type=jax.ShapeDtypeStruct((batch, seq, feat), jnp.float32),
        mesh=plsc.VectorSubcoreMesh(core_axis_name="c", subcore_axis_name="s"),
        scratch_types=[
            pltpu.VMEM((bpw, seq), jnp.int32),
            pltpu.VMEM((NBUF, NBS, seq, feat), jnp.float32),
            pltpu.SemaphoreType.DMA((NBUF,)),
            pltpu.SemaphoreType.DMA((NBUF,)),
        ],
        compiler_params=pltpu.CompilerParams(use_tc_tiling_on_sc=False),
    )


def kernel(inputs, embedding):
    batch, seq = inputs.shape
    _, feat = embedding.shape
    return _build(batch, seq, feat)(inputs, embedding)
